# baseline (speedup 1.0000x reference)
import jax
import jax.numpy as jnp
from jax import lax
from jax.experimental import pallas as pl
from jax.experimental.pallas import tpu as pltpu
from jax.experimental.pallas import tpu_sc as plsc

T, D, H, E = 2048, 768, 256, 64
TOPK = 2
P = T * TOPK
BLK = 128
NBLK = P // BLK + E
NP = NBLK * BLK
CH = 128

NC, NS = 2, 16
NW = NC * NS
PPW = P // NW
TPW = T // NW
CHT = 32


def _bf16_round_hi16(u):
    return lax.shift_right_logical(
        u + 0x7FFF + (lax.shift_right_logical(u, 16) & 1), 16)


def _gating_body(x_ref, wg_ref, bg_ref,
                 gates_ref, idx_ref, gp_ref, pp_ref, eob_ref,
                 loss_ref, util_ref, xpack_ref):
    x = x_ref[...]
    v = lax.bitcast_convert_type(x, jnp.int32)
    lo = _bf16_round_hi16(v[:, :D // 2])
    hi = _bf16_round_hi16(v[:, D // 2:])
    xpack_ref[...] = lax.shift_left(hi, 16) | lo
    logits = jnp.dot(x, wg_ref[...], preferred_element_type=jnp.float32)
    logits = logits + bg_ref[...]

    iota_e = lax.broadcasted_iota(jnp.int32, (T, E), 1)
    m1 = jnp.max(logits, axis=1, keepdims=True)
    i1 = jnp.min(jnp.where(logits == m1, iota_e, E), axis=1, keepdims=True)
    mask1 = iota_e == i1
    logits2 = jnp.where(mask1, -jnp.inf, logits)
    m2 = jnp.max(logits2, axis=1, keepdims=True)
    i2 = jnp.min(jnp.where(logits2 == m2, iota_e, E), axis=1, keepdims=True)
    mask2 = iota_e == i2

    e2 = jnp.exp(m2 - m1)
    denom = 1.0 + e2
    g1 = 1.0 / denom
    g2 = e2 / denom
    gates = jnp.where(mask1, g1, 0.0) + jnp.where(mask2, g2, 0.0)
    gates_ref[...] = gates

    ez = jnp.exp(logits - m1)
    sm = ez / jnp.sum(ez, axis=1, keepdims=True)
    importance = jnp.sum(gates, axis=0, keepdims=True)
    load = jnp.sum(sm, axis=0, keepdims=True)

    def _cv(v):
        mean = jnp.sum(v) / E
        var = jnp.sum((v - mean) ** 2) / (E - 1)
        return jnp.sqrt(var) / (mean + 1e-6)

    loss_ref[...] = jnp.reshape(_cv(importance) + _cv(load), (1, 1))
    util_ref[...] = jnp.sum((gates > 0).astype(jnp.float32), axis=0,
                            keepdims=True) / T

    sel = (mask1 | mask2).astype(jnp.float32)
    counts = jnp.sum(sel, axis=0, keepdims=True)
    r_io = lax.broadcasted_iota(jnp.int32, (CH, CH), 0)
    c_io = lax.broadcasted_iota(jnp.int32, (CH, CH), 1)
    tril = (r_io > c_io).astype(jnp.float32)
    running = jnp.zeros((1, E), jnp.float32)
    ranks = []
    for c in range(T // CH):
        chunk = sel[c * CH:(c + 1) * CH, :]
        ranks.append(jnp.dot(tril, chunk,
                             preferred_element_type=jnp.float32) + running)
        running = running + jnp.sum(chunk, axis=0, keepdims=True)
    rank = jnp.concatenate(ranks, axis=0)

    ci = counts.astype(jnp.int32)
    pc = ((ci + (BLK - 1)) // BLK) * BLK
    pcf = pc.astype(jnp.float32)
    er = lax.broadcasted_iota(jnp.int32, (E, E), 0)
    ec = lax.broadcasted_iota(jnp.int32, (E, E), 1)
    upper = (er < ec).astype(jnp.float32)
    offs = jnp.dot(pcf, upper, preferred_element_type=jnp.float32)
    cum_incl = offs + pcf

    posf = offs + rank
    pos1 = jnp.sum(jnp.where(mask1, posf, 0.0), axis=1, keepdims=True)
    pos2 = jnp.sum(jnp.where(mask2, posf, 0.0), axis=1, keepdims=True)

    lane = lax.broadcasted_iota(jnp.int32, (T, 128), 1)
    idx_ref[...] = jnp.where(lane == 0, i1, jnp.where(lane == 1, i2, 0))
    pp_ref[...] = jnp.where(lane == 0, pos1.astype(jnp.int32),
                            jnp.where(lane == 1, pos2.astype(jnp.int32), 0))
    gp_ref[...] = jnp.where(lane == 0, g1, jnp.where(lane == 1, g2, 0.0))

    bi = lax.broadcasted_iota(jnp.int32, (NBLK, E), 0)
    rstart = (bi * BLK).astype(jnp.float32)
    cumb = jnp.broadcast_to(cum_incl, (NBLK, E))
    eobf = jnp.sum((cumb <= rstart).astype(jnp.float32), axis=1,
                   keepdims=True)
    eob_i = jnp.minimum(eobf.astype(jnp.int32), E - 1)
    total = jnp.sum(pcf)
    used_i = (rstart[:, :1] < total).astype(jnp.int32)
    lane_b = lax.broadcasted_iota(jnp.int32, (NBLK, 128), 1)
    eob_ref[...] = jnp.where(lane_b == 1, used_i, eob_i)


def _gating_call(x, Wg, bg):
    return pl.pallas_call(
        _gating_body,
        out_shape=[
            jax.ShapeDtypeStruct((T, E), jnp.float32),
            jax.ShapeDtypeStruct((T, 128), jnp.int32),
            jax.ShapeDtypeStruct((T, 128), jnp.float32),
            jax.ShapeDtypeStruct((T, 128), jnp.int32),
            jax.ShapeDtypeStruct((NBLK, 128), jnp.int32),
            jax.ShapeDtypeStruct((1, 1), jnp.float32),
            jax.ShapeDtypeStruct((1, E), jnp.float32),
            jax.ShapeDtypeStruct((T, D // 2), jnp.int32),
        ],
    )(x, Wg, bg.reshape(1, E))


NCH = 4
DCH = PPW // NCH


def _dispatch_body(x_hbm, posq_hbm, xs_hbm, idx_v, pos_v, rows_v,
                   sem_g, sem_s):
    wid = lax.axis_index("s") * NC + lax.axis_index("c")
    base = wid * PPW
    for c4 in range(NCH):
        for c in range(DCH // 16):
            pair = base + c4 * DCH + c * 16 + lax.iota(jnp.int32, 16)
            idx_v[c4, pl.ds(c * 16, 16)] = lax.shift_right_logical(pair, 1)
    pltpu.sync_copy(posq_hbm.at[wid], pos_v)
    gathers = [
        pltpu.async_copy(x_hbm.at[idx_v.at[c4]],
                         rows_v.at[pl.ds(c4 * DCH, DCH)], sem_g)
        for c4 in range(NCH)
    ]
    scatters = []
    for c4 in range(NCH):
        gathers[c4].wait()
        scatters.append(
            pltpu.async_copy(rows_v.at[pl.ds(c4 * DCH, DCH)],
                             xs_hbm.at[pos_v.at[c4]], sem_s))
    for cp in scatters:
        cp.wait()


def _dispatch_call(x_rows_i32, pos_q):
    mesh = plsc.VectorSubcoreMesh(core_axis_name="c", subcore_axis_name="s")
    return pl.kernel(
        _dispatch_body,
        mesh=mesh,
        out_type=jax.ShapeDtypeStruct((NP, D // 2), jnp.int32),
        scratch_types=[
            pltpu.VMEM((NCH, DCH), jnp.int32),
            pltpu.VMEM((NCH, DCH), jnp.int32),
            pltpu.VMEM((PPW, D // 2), jnp.int32),
            pltpu.SemaphoreType.DMA,
            pltpu.SemaphoreType.DMA,
        ],
    )(x_rows_i32, pos_q)


def _ffn_body(eob_ref, used_ref, xs_ref, w1a_ref, w1b_ref, b1_ref, w2_ref,
              b2_ref, ys_ref):
    blk = pl.program_id(0)

    @pl.when(used_ref[blk] == 1)
    def _():
        xi = xs_ref[...]
        a = lax.bitcast_convert_type(lax.shift_left(xi, 16), jnp.float32)
        b = lax.bitcast_convert_type(xi & jnp.int32(-65536), jnp.float32)
        h = (jnp.dot(a, w1a_ref[0], preferred_element_type=jnp.float32)
             + jnp.dot(b, w1b_ref[0], preferred_element_type=jnp.float32))
        h = h + b1_ref[0]
        h = 0.5 * h * (1.0 + lax.erf(h * 0.7071067811865476))
        y = jnp.dot(h, w2_ref[0], preferred_element_type=jnp.float32)
        y = y + b2_ref[0]
        ylo = y[:, :D // 2]
        yhi = y[:, D // 2:]
        plo = _bf16_round_hi16(lax.bitcast_convert_type(ylo, jnp.int32))
        phi = _bf16_round_hi16(lax.bitcast_convert_type(yhi, jnp.int32))
        ys_ref[...] = lax.shift_left(phi, 16) | plo


def _ffn_call(eob, used, xs, W1, b1, W2, b2):
    grid_spec = pltpu.PrefetchScalarGridSpec(
        num_scalar_prefetch=2,
        grid=(NBLK,),
        in_specs=[
            pl.BlockSpec((BLK, D // 2),
                         lambda b, eob, used: (used[b] * b, 0)),
            pl.BlockSpec((1, D // 2, H),
                         lambda b, eob, used: (eob[b], 0, 0)),
            pl.BlockSpec((1, D // 2, H),
                         lambda b, eob, used: (eob[b], 1, 0)),
            pl.BlockSpec((1, 1, H), lambda b, eob, used: (eob[b], 0, 0)),
            pl.BlockSpec((1, H, D), lambda b, eob, used: (eob[b], 0, 0)),
            pl.BlockSpec((1, 1, D), lambda b, eob, used: (eob[b], 0, 0)),
        ],
        out_specs=pl.BlockSpec(
            (BLK, D // 2),
            lambda b, eob, used: (jnp.where(used[b] == 1, b, NBLK), 0)),
    )
    return pl.pallas_call(
        _ffn_body,
        grid_spec=grid_spec,
        out_shape=jax.ShapeDtypeStruct(((NBLK + 1) * BLK, D // 2), jnp.int32),
    )(eob, used, xs, W1, W1, b1.reshape(E, 1, H), W2, b2.reshape(E, 1, D))


def _lane_splat(vec, j):
    dnums = lax.GatherDimensionNumbers(
        offset_dims=(), collapsed_slice_dims=(0,), start_index_map=(0,))
    starts = jnp.full((16, 1), j, jnp.int32)
    return lax.gather(vec, starts, dnums, slice_sizes=(1,),
                      mode=lax.GatherScatterMode.PROMISE_IN_BOUNDS)


def _combine_body(ys_hbm, p1_hbm, p2_hbm, g1_hbm, g2_hbm, out_hbm,
                  p1_v, p2_v, g1_v, g2_v, a0_v, b0_v, a1_v, b1_v, o_v, sem,
                  sem_o):
    wid = lax.axis_index("s") * NC + lax.axis_index("c")
    base = wid * TPW
    pltpu.sync_copy(p1_hbm.at[pl.ds(base, TPW)], p1_v)
    pltpu.sync_copy(p2_hbm.at[pl.ds(base, TPW)], p2_v)
    pltpu.sync_copy(g1_hbm.at[pl.ds(base, TPW)], g1_v)
    pltpu.sync_copy(g2_hbm.at[pl.ds(base, TPW)], g2_v)
    cps = []
    for hh, (a_v, b_v) in enumerate(((a0_v, b0_v), (a1_v, b1_v))):
        sl_t = pl.ds(hh * CHT, CHT)
        cps.append((pltpu.async_copy(ys_hbm.at[p1_v.at[sl_t]], a_v, sem),
                    pltpu.async_copy(ys_hbm.at[p2_v.at[sl_t]], b_v, sem)))
    mask_hi = jnp.int32(-65536)
    stores = []
    for hh, (a_v, b_v) in enumerate(((a0_v, b0_v), (a1_v, b1_v))):
        cps[hh][0].wait()
        cps[hh][1].wait()
        for tg in range(CHT // 16):
            gv1 = g1_v[pl.ds(hh * CHT + tg * 16, 16)]
            gv2 = g2_v[pl.ds(hh * CHT + tg * 16, 16)]
            for j in range(16):
                g1b = _lane_splat(gv1, j)
                g2b = _lane_splat(gv2, j)
                row = tg * 16 + j
                orow = hh * CHT + row

                def _col(ci, _, row=row, orow=orow, g1b=g1b, g2b=g2b,
                         a_v=a_v, b_v=b_v):
                    sl = pl.ds(ci * 16, 16)
                    ai = a_v[row, sl]
                    bi = b_v[row, sl]
                    alo = lax.bitcast_convert_type(lax.shift_left(ai, 16),
                                                   jnp.float32)
                    blo = lax.bitcast_convert_type(lax.shift_left(bi, 16),
                                                   jnp.float32)
                    ahi = lax.bitcast_convert_type(ai & mask_hi, jnp.float32)
                    bhi = lax.bitcast_convert_type(bi & mask_hi, jnp.float32)
                    o_v[orow, sl] = g1b * alo + g2b * blo
                    o_v[orow, pl.ds(D // 2 + ci * 16, 16)] = (g1b * ahi
                                                              + g2b * bhi)
                    return _

                lax.fori_loop(0, D // 32, _col, None)
        stores.append(
            pltpu.async_copy(o_v.at[pl.ds(hh * CHT, CHT)],
                             out_hbm.at[pl.ds(base + hh * CHT, CHT)], sem_o))
    for cp in stores:
        cp.wait()


def _combine_call(ys, p1, p2, g1, g2):
    mesh = plsc.VectorSubcoreMesh(core_axis_name="c", subcore_axis_name="s")
    return pl.kernel(
        _combine_body,
        mesh=mesh,
        out_type=jax.ShapeDtypeStruct((T, D), jnp.float32),
        scratch_types=[
            pltpu.VMEM((TPW,), jnp.int32),
            pltpu.VMEM((TPW,), jnp.int32),
            pltpu.VMEM((TPW,), jnp.float32),
            pltpu.VMEM((TPW,), jnp.float32),
            pltpu.VMEM((CHT, D // 2), jnp.int32),
            pltpu.VMEM((CHT, D // 2), jnp.int32),
            pltpu.VMEM((CHT, D // 2), jnp.int32),
            pltpu.VMEM((CHT, D // 2), jnp.int32),
            pltpu.VMEM((TPW, D), jnp.float32),
            pltpu.SemaphoreType.DMA,
            pltpu.SemaphoreType.DMA,
        ],
    )(ys, p1, p2, g1, g2)


def kernel(x, Wg, bg, W1, b1, W2, b2):
    gates, idx_pad, gp_pad, pp_pad, eob_pad, loss2d, util2d, xpack = \
        _gating_call(x, Wg, bg)
    top_i = idx_pad[:, :TOPK]
    pos_q = pp_pad[:, :TOPK].reshape(NW, NCH, DCH)
    eob = eob_pad[:, 0]
    used = eob_pad[:, 1]
    p1, p2 = pp_pad[:, 0], pp_pad[:, 1]

    xs_i32 = _dispatch_call(xpack, pos_q)
    ys = _ffn_call(eob, used, xs_i32, W1, b1, W2, b2)
    out = _combine_call(ys, p1, p2, gp_pad[:, 0], gp_pad[:, 1])

    return (out, gates, top_i, loss2d[0, 0], util2d[0])


# speedup vs baseline: 1.0066x; 1.0066x over previous
import jax
import jax.numpy as jnp
from jax import lax
from jax.experimental import pallas as pl
from jax.experimental.pallas import tpu as pltpu
from jax.experimental.pallas import tpu_sc as plsc

T, D, H, E = 2048, 768, 256, 64
TOPK = 2
P = T * TOPK
BLK = 128
NBLK = P // BLK + E
NP = NBLK * BLK
CH = 128

NC, NS = 2, 16
NW = NC * NS
PPW = P // NW
TPW = T // NW
CHT = 32


def _bf16_round_hi16(u):
    return lax.shift_right_logical(
        u + 0x7FFF + (lax.shift_right_logical(u, 16) & 1), 16)


def _gating_body(x_ref, wg_ref, bg_ref,
                 gates_ref, idx_ref, gp_ref, pp_ref, eob_ref,
                 loss_ref, util_ref, xpack_ref):
    x = x_ref[...]
    v = lax.bitcast_convert_type(x, jnp.int32)
    lo = _bf16_round_hi16(v[:, :D // 2])
    hi = _bf16_round_hi16(v[:, D // 2:])
    xpack_ref[...] = lax.shift_left(hi, 16) | lo
    logits = jnp.dot(x, wg_ref[...], preferred_element_type=jnp.float32)
    logits = logits + bg_ref[...]

    iota_e = lax.broadcasted_iota(jnp.int32, (T, E), 1)
    m1 = jnp.max(logits, axis=1, keepdims=True)
    i1 = jnp.min(jnp.where(logits == m1, iota_e, E), axis=1, keepdims=True)
    mask1 = iota_e == i1
    logits2 = jnp.where(mask1, -jnp.inf, logits)
    m2 = jnp.max(logits2, axis=1, keepdims=True)
    i2 = jnp.min(jnp.where(logits2 == m2, iota_e, E), axis=1, keepdims=True)
    mask2 = iota_e == i2

    e2 = jnp.exp(m2 - m1)
    denom = 1.0 + e2
    g1 = 1.0 / denom
    g2 = e2 / denom
    gates = jnp.where(mask1, g1, 0.0) + jnp.where(mask2, g2, 0.0)
    gates_ref[...] = gates

    ez = jnp.exp(logits - m1)
    sm = ez / jnp.sum(ez, axis=1, keepdims=True)
    importance = jnp.sum(gates, axis=0, keepdims=True)
    load = jnp.sum(sm, axis=0, keepdims=True)

    def _cv(v):
        mean = jnp.sum(v) / E
        var = jnp.sum((v - mean) ** 2) / (E - 1)
        return jnp.sqrt(var) / (mean + 1e-6)

    loss_ref[...] = jnp.reshape(_cv(importance) + _cv(load), (1, 1))
    util_ref[...] = jnp.sum((gates > 0).astype(jnp.float32), axis=0,
                            keepdims=True) / T

    sel = (mask1 | mask2).astype(jnp.float32)
    counts = jnp.sum(sel, axis=0, keepdims=True)
    r_io = lax.broadcasted_iota(jnp.int32, (CH, CH), 0)
    c_io = lax.broadcasted_iota(jnp.int32, (CH, CH), 1)
    tril = (r_io > c_io).astype(jnp.float32)
    running = jnp.zeros((1, E), jnp.float32)
    ranks = []
    for c in range(T // CH):
        chunk = sel[c * CH:(c + 1) * CH, :]
        ranks.append(jnp.dot(tril, chunk,
                             preferred_element_type=jnp.float32) + running)
        running = running + jnp.sum(chunk, axis=0, keepdims=True)
    rank = jnp.concatenate(ranks, axis=0)

    ci = counts.astype(jnp.int32)
    pc = ((ci + (BLK - 1)) // BLK) * BLK
    pcf = pc.astype(jnp.float32)
    er = lax.broadcasted_iota(jnp.int32, (E, E), 0)
    ec = lax.broadcasted_iota(jnp.int32, (E, E), 1)
    upper = (er < ec).astype(jnp.float32)
    offs = jnp.dot(pcf, upper, preferred_element_type=jnp.float32)
    cum_incl = offs + pcf

    posf = offs + rank
    pos1 = jnp.sum(jnp.where(mask1, posf, 0.0), axis=1, keepdims=True)
    pos2 = jnp.sum(jnp.where(mask2, posf, 0.0), axis=1, keepdims=True)

    lane = lax.broadcasted_iota(jnp.int32, (T, 128), 1)
    idx_ref[...] = jnp.where(lane == 0, i1, jnp.where(lane == 1, i2, 0))
    pp_ref[...] = jnp.where(lane == 0, pos1.astype(jnp.int32),
                            jnp.where(lane == 1, pos2.astype(jnp.int32), 0))
    gp_ref[...] = jnp.where(lane == 0, g1, jnp.where(lane == 1, g2, 0.0))

    bi = lax.broadcasted_iota(jnp.int32, (NBLK, E), 0)
    rstart = (bi * BLK).astype(jnp.float32)
    cumb = jnp.broadcast_to(cum_incl, (NBLK, E))
    eobf = jnp.sum((cumb <= rstart).astype(jnp.float32), axis=1,
                   keepdims=True)
    eob_i = jnp.minimum(eobf.astype(jnp.int32), E - 1)
    total = jnp.sum(pcf)
    used_i = (rstart[:, :1] < total).astype(jnp.int32)
    lane_b = lax.broadcasted_iota(jnp.int32, (NBLK, 128), 1)
    eob_ref[...] = jnp.where(lane_b == 1, used_i, eob_i)


def _gating_call(x, Wg, bg):
    return pl.pallas_call(
        _gating_body,
        out_shape=[
            jax.ShapeDtypeStruct((T, E), jnp.float32),
            jax.ShapeDtypeStruct((T, 128), jnp.int32),
            jax.ShapeDtypeStruct((T, 128), jnp.float32),
            jax.ShapeDtypeStruct((T, 128), jnp.int32),
            jax.ShapeDtypeStruct((NBLK, 128), jnp.int32),
            jax.ShapeDtypeStruct((1, 1), jnp.float32),
            jax.ShapeDtypeStruct((1, E), jnp.float32),
            jax.ShapeDtypeStruct((T, D // 2), jnp.int32),
        ],
    )(x, Wg, bg.reshape(1, E))


NCH = 4
DCH = PPW // NCH


def _dispatch_body(x_hbm, posq_hbm, xs_hbm, idx_v, pos_v, rows_v,
                   sem_g, sem_s):
    wid = lax.axis_index("s") * NC + lax.axis_index("c")
    base = wid * PPW
    for c4 in range(NCH):
        for c in range(DCH // 16):
            pair = base + c4 * DCH + c * 16 + lax.iota(jnp.int32, 16)
            idx_v[c4, pl.ds(c * 16, 16)] = lax.shift_right_logical(pair, 1)
    pltpu.sync_copy(posq_hbm.at[wid], pos_v)
    gathers = [
        pltpu.async_copy(x_hbm.at[idx_v.at[c4]],
                         rows_v.at[pl.ds(c4 * DCH, DCH)], sem_g)
        for c4 in range(NCH)
    ]
    scatters = []
    for c4 in range(NCH):
        gathers[c4].wait()
        scatters.append(
            pltpu.async_copy(rows_v.at[pl.ds(c4 * DCH, DCH)],
                             xs_hbm.at[pos_v.at[c4]], sem_s))
    for cp in scatters:
        cp.wait()


def _dispatch_call(x_rows_i32, pos_q):
    mesh = plsc.VectorSubcoreMesh(core_axis_name="c", subcore_axis_name="s")
    return pl.kernel(
        _dispatch_body,
        mesh=mesh,
        out_type=jax.ShapeDtypeStruct((NP, D // 2), jnp.int32),
        scratch_types=[
            pltpu.VMEM((NCH, DCH), jnp.int32),
            pltpu.VMEM((NCH, DCH), jnp.int32),
            pltpu.VMEM((PPW, D // 2), jnp.int32),
            pltpu.SemaphoreType.DMA,
            pltpu.SemaphoreType.DMA,
        ],
    )(x_rows_i32, pos_q)


def _ffn_body(eob_ref, used_ref, xs_ref, w1_ref, b1_ref, w2_ref, b2_ref,
              ys_ref):
    blk = pl.program_id(0)

    @pl.when(used_ref[blk] == 1)
    def _():
        xi = xs_ref[...]
        a = lax.bitcast_convert_type(lax.shift_left(xi, 16), jnp.float32)
        b = lax.bitcast_convert_type(xi & jnp.int32(-65536), jnp.float32)
        h = (jnp.dot(a, w1_ref[0, :D // 2, :],
                     preferred_element_type=jnp.float32)
             + jnp.dot(b, w1_ref[0, D // 2:, :],
                       preferred_element_type=jnp.float32))
        h = h + b1_ref[0]
        h = 0.5 * h * (1.0 + lax.erf(h * 0.7071067811865476))
        y = jnp.dot(h, w2_ref[0], preferred_element_type=jnp.float32)
        y = y + b2_ref[0]
        ylo = y[:, :D // 2]
        yhi = y[:, D // 2:]
        plo = _bf16_round_hi16(lax.bitcast_convert_type(ylo, jnp.int32))
        phi = _bf16_round_hi16(lax.bitcast_convert_type(yhi, jnp.int32))
        ys_ref[...] = lax.shift_left(phi, 16) | plo


def _ffn_call(eob, used, xs, W1, b1, W2, b2):
    grid_spec = pltpu.PrefetchScalarGridSpec(
        num_scalar_prefetch=2,
        grid=(NBLK,),
        in_specs=[
            pl.BlockSpec((BLK, D // 2),
                         lambda b, eob, used: (used[b] * b, 0)),
            pl.BlockSpec((1, D, H), lambda b, eob, used: (eob[b], 0, 0)),
            pl.BlockSpec((1, 1, H), lambda b, eob, used: (eob[b], 0, 0)),
            pl.BlockSpec((1, H, D), lambda b, eob, used: (eob[b], 0, 0)),
            pl.BlockSpec((1, 1, D), lambda b, eob, used: (eob[b], 0, 0)),
        ],
        out_specs=pl.BlockSpec(
            (BLK, D // 2),
            lambda b, eob, used: (jnp.where(used[b] == 1, b, NBLK), 0)),
    )
    return pl.pallas_call(
        _ffn_body,
        grid_spec=grid_spec,
        out_shape=jax.ShapeDtypeStruct(((NBLK + 1) * BLK, D // 2), jnp.int32),
    )(eob, used, xs, W1, b1.reshape(E, 1, H), W2, b2.reshape(E, 1, D))


def _lane_splat(vec, j):
    dnums = lax.GatherDimensionNumbers(
        offset_dims=(), collapsed_slice_dims=(0,), start_index_map=(0,))
    starts = jnp.full((16, 1), j, jnp.int32)
    return lax.gather(vec, starts, dnums, slice_sizes=(1,),
                      mode=lax.GatherScatterMode.PROMISE_IN_BOUNDS)


def _combine_body(ys_hbm, p1_hbm, p2_hbm, g1_hbm, g2_hbm, out_hbm,
                  p1_v, p2_v, g1_v, g2_v, a0_v, b0_v, a1_v, b1_v, o_v, sem,
                  sem_o):
    wid = lax.axis_index("s") * NC + lax.axis_index("c")
    base = wid * TPW
    pltpu.sync_copy(p1_hbm.at[pl.ds(base, TPW)], p1_v)
    pltpu.sync_copy(p2_hbm.at[pl.ds(base, TPW)], p2_v)
    pltpu.sync_copy(g1_hbm.at[pl.ds(base, TPW)], g1_v)
    pltpu.sync_copy(g2_hbm.at[pl.ds(base, TPW)], g2_v)
    cps = []
    for hh, (a_v, b_v) in enumerate(((a0_v, b0_v), (a1_v, b1_v))):
        sl_t = pl.ds(hh * CHT, CHT)
        cps.append((pltpu.async_copy(ys_hbm.at[p1_v.at[sl_t]], a_v, sem),
                    pltpu.async_copy(ys_hbm.at[p2_v.at[sl_t]], b_v, sem)))
    mask_hi = jnp.int32(-65536)
    stores = []
    for hh, (a_v, b_v) in enumerate(((a0_v, b0_v), (a1_v, b1_v))):
        cps[hh][0].wait()
        cps[hh][1].wait()
        for tg in range(CHT // 16):
            gv1 = g1_v[pl.ds(hh * CHT + tg * 16, 16)]
            gv2 = g2_v[pl.ds(hh * CHT + tg * 16, 16)]
            for j in range(16):
                g1b = _lane_splat(gv1, j)
                g2b = _lane_splat(gv2, j)
                row = tg * 16 + j
                orow = hh * CHT + row

                def _col(ci, _, row=row, orow=orow, g1b=g1b, g2b=g2b,
                         a_v=a_v, b_v=b_v):
                    sl = pl.ds(ci * 16, 16)
                    ai = a_v[row, sl]
                    bi = b_v[row, sl]
                    alo = lax.bitcast_convert_type(lax.shift_left(ai, 16),
                                                   jnp.float32)
                    blo = lax.bitcast_convert_type(lax.shift_left(bi, 16),
                                                   jnp.float32)
                    ahi = lax.bitcast_convert_type(ai & mask_hi, jnp.float32)
                    bhi = lax.bitcast_convert_type(bi & mask_hi, jnp.float32)
                    o_v[orow, sl] = g1b * alo + g2b * blo
                    o_v[orow, pl.ds(D // 2 + ci * 16, 16)] = (g1b * ahi
                                                              + g2b * bhi)
                    return _

                lax.fori_loop(0, D // 32, _col, None)
        stores.append(
            pltpu.async_copy(o_v.at[pl.ds(hh * CHT, CHT)],
                             out_hbm.at[pl.ds(base + hh * CHT, CHT)], sem_o))
    for cp in stores:
        cp.wait()


def _combine_call(ys, p1, p2, g1, g2):
    mesh = plsc.VectorSubcoreMesh(core_axis_name="c", subcore_axis_name="s")
    return pl.kernel(
        _combine_body,
        mesh=mesh,
        out_type=jax.ShapeDtypeStruct((T, D), jnp.float32),
        scratch_types=[
            pltpu.VMEM((TPW,), jnp.int32),
            pltpu.VMEM((TPW,), jnp.int32),
            pltpu.VMEM((TPW,), jnp.float32),
            pltpu.VMEM((TPW,), jnp.float32),
            pltpu.VMEM((CHT, D // 2), jnp.int32),
            pltpu.VMEM((CHT, D // 2), jnp.int32),
            pltpu.VMEM((CHT, D // 2), jnp.int32),
            pltpu.VMEM((CHT, D // 2), jnp.int32),
            pltpu.VMEM((TPW, D), jnp.float32),
            pltpu.SemaphoreType.DMA,
            pltpu.SemaphoreType.DMA,
        ],
    )(ys, p1, p2, g1, g2)


def kernel(x, Wg, bg, W1, b1, W2, b2):
    gates, idx_pad, gp_pad, pp_pad, eob_pad, loss2d, util2d, xpack = \
        _gating_call(x, Wg, bg)
    top_i = idx_pad[:, :TOPK]
    pos_q = pp_pad[:, :TOPK].reshape(NW, NCH, DCH)
    eob = eob_pad[:, 0]
    used = eob_pad[:, 1]
    p1, p2 = pp_pad[:, 0], pp_pad[:, 1]

    xs_i32 = _dispatch_call(xpack, pos_q)
    ys = _ffn_call(eob, used, xs_i32, W1, b1, W2, b2)
    out = _combine_call(ys, p1, p2, gp_pad[:, 0], gp_pad[:, 1])

    return (out, gates, top_i, loss2d[0, 0], util2d[0])
